# baseline (speedup 1.0000x reference)
import functools

import jax
import jax.numpy as jnp
from jax import lax
from jax.experimental import pallas as pl
from jax.experimental.pallas import tpu as pltpu
from jax.experimental.pallas import tpu_sc as plsc

B, S, L = 1024, 20, 20
T = S * L
VOCAB, D, NCLASS = 100000, 128, 100
DV = D // 16

NC, NS = 2, 16
NW = NC * NS
BPW = B // NW
CH = 5
CHN = T // CH
GH = 8
NG = BPW // GH

_mesh = plsc.VectorSubcoreMesh(core_axis_name="c", subcore_axis_name="s")


@functools.partial(
    pl.kernel,
    mesh=_mesh,
    out_type=jax.ShapeDtypeStruct((B, D), jnp.float32),
    scratch_types=[
        pltpu.VMEM((2, GH, CH, CHN), jnp.int32),
        pltpu.VMEM((10, CHN, D), jnp.float32),
        pltpu.VMEM((BPW, D), jnp.float32),
        [pltpu.SemaphoreType.DMA] * 10,
        pltpu.SemaphoreType.DMA,
    ],
)
def _pool(x_hbm, table_hbm, out_hbm, idx_v, rows_v, h_v, sems, isem):
    wid = lax.axis_index("s") * NC + lax.axis_index("c")
    base = wid * BPW

    def fire(qr, buf):
        rb = qr // CH
        g = rb // GH
        i = rb - g * GH
        pltpu.async_copy(
            table_hbm.at[idx_v.at[g & 1, i, qr % CH]],
            rows_v.at[buf],
            sems[buf],
        )

    def drain(buf):
        pltpu.make_async_copy(
            table_hbm.at[pl.ds(0, CHN)], rows_v.at[buf], sems[buf]
        ).wait()

    def reduce_q(buf, accs):
        def body(r, accs):
            return tuple(
                jnp.maximum(a, rows_v[buf, r, pl.ds(16 * d, 16)])
                for d, a in enumerate(accs)
            )
        return lax.fori_loop(0, CHN, body, accs, unroll=8)

    pltpu.sync_copy(x_hbm.at[pl.ds(base, GH)], idx_v.at[0])
    pltpu.async_copy(x_hbm.at[pl.ds(base + GH, GH)], idx_v.at[1], isem)

    for k in range(10):
        fire(k, k)

    def body(g, carry):
        for k in range(10):
            qr = 10 * g + k
            drain(k)
            if k % CH == 0:
                accs = tuple(
                    jnp.full((16,), -jnp.inf, jnp.float32) for _ in range(DV)
                )
            accs = reduce_q(k, accs)
            if k % CH == CH - 1:
                b = 2 * g + k // CH
                for d in range(DV):
                    h_v[b, pl.ds(16 * d, 16)] = accs[d]
            nq = qr + 10
            if k == 0:
                nrb = nq // CH

                @pl.when(jnp.logical_and(nrb % GH == 0, nq < CH * BPW))
                def _():
                    pltpu.make_async_copy(
                        x_hbm.at[pl.ds(0, GH)], idx_v.at[0], isem
                    ).wait()
                    gg = nrb // GH + 1

                    @pl.when(gg < NG)
                    def _():
                        pltpu.async_copy(
                            x_hbm.at[pl.ds(base + gg * GH, GH)],
                            idx_v.at[gg & 1],
                            isem,
                        )

            @pl.when(nq < CH * BPW)
            def _():
                fire(nq, k)
        return carry

    lax.fori_loop(0, BPW // 2, body, 0)

    pltpu.sync_copy(h_v, out_hbm.at[pl.ds(base, BPW)])


def _fc_body(h_ref, w_ref, b_ref, o_ref):
    logits = lax.dot_general(
        h_ref[...], w_ref[...], (((1,), (1,)), ((), ())),
        preferred_element_type=jnp.float32,
    )
    o_ref[...] = jax.nn.sigmoid(logits + b_ref[...])


def _fc(h, W, b):
    return pl.pallas_call(
        _fc_body,
        out_shape=jax.ShapeDtypeStruct((B, NCLASS), jnp.float32),
    )(h, W, b.reshape(1, NCLASS))


def kernel(x, table, W, b):
    xi = x.astype(jnp.int32).reshape(B, CH, CHN)
    h = _pool(xi, table)
    return _fc(h, W, b)


# speedup vs baseline: 1.0220x; 1.0220x over previous
import functools

import jax
import jax.numpy as jnp
from jax import lax
from jax.experimental import pallas as pl
from jax.experimental.pallas import tpu as pltpu
from jax.experimental.pallas import tpu_sc as plsc

B, S, L = 1024, 20, 20
T = S * L
VOCAB, D, NCLASS = 100000, 128, 100
DV = D // 16

NC, NS = 2, 16
NW = NC * NS
BPW = B // NW
CH = 4
CHN = T // CH
GH = 8
NG = BPW // GH

_mesh = plsc.VectorSubcoreMesh(core_axis_name="c", subcore_axis_name="s")


@functools.partial(
    pl.kernel,
    mesh=_mesh,
    out_type=jax.ShapeDtypeStruct((B, D), jnp.float32),
    scratch_types=[
        pltpu.VMEM((2, GH, CH, CHN), jnp.int32),
        pltpu.VMEM((4, T // 2, D), jnp.float32),
        pltpu.VMEM((BPW, D), jnp.float32),
        pltpu.SemaphoreType.DMA,
        pltpu.SemaphoreType.DMA,
        pltpu.SemaphoreType.DMA,
        pltpu.SemaphoreType.DMA,
        pltpu.SemaphoreType.DMA,
    ],
)
def _pool(x_hbm, table_hbm, out_hbm, idx_v, rows_v, h_v,
          sem0, sem1, sem2, sem3, isem):
    wid = lax.axis_index("s") * NC + lax.axis_index("c")
    base = wid * BPW
    sems = (sem0, sem1, sem2, sem3)
    HR = T // 2
    HCH = CH // 2

    def fire(hr, buf):
        rb = hr // 2
        g = rb // GH
        i = rb - g * GH
        for c in range(HCH):
            pltpu.async_copy(
                table_hbm.at[idx_v.at[g & 1, i, (hr % 2) * HCH + c]],
                rows_v.at[buf, pl.ds(c * CHN, CHN)],
                sems[buf],
            )

    def drain(buf):
        pltpu.make_async_copy(
            table_hbm.at[pl.ds(0, HR)], rows_v.at[buf], sems[buf]
        ).wait()

    def reduce_half(buf, accs):
        def body(r, accs):
            return tuple(
                jnp.maximum(a, rows_v[buf, r, pl.ds(16 * d, 16)])
                for d, a in enumerate(accs)
            )
        return lax.fori_loop(0, HR, body, accs, unroll=8)

    pltpu.sync_copy(x_hbm.at[pl.ds(base, GH)], idx_v.at[0])
    pltpu.async_copy(x_hbm.at[pl.ds(base + GH, GH)], idx_v.at[1], isem)

    for k in range(4):
        fire(k, k)

    def body(g, carry):
        for k in range(4):
            hr = 4 * g + k
            drain(k)
            if k % 2 == 0:
                accs = tuple(
                    jnp.full((16,), -jnp.inf, jnp.float32) for _ in range(DV)
                )
            accs = reduce_half(k, accs)
            if k % 2 == 1:
                b = 2 * g + k // 2
                for d in range(DV):
                    h_v[b, pl.ds(16 * d, 16)] = accs[d]
            nh = hr + 4
            if k % 2 == 0:
                nrb = nh // 2

                @pl.when(jnp.logical_and(nrb % GH == 0,
                                         jnp.logical_and(nh % 2 == 0,
                                                         nh < 2 * BPW)))
                def _():
                    pltpu.make_async_copy(
                        x_hbm.at[pl.ds(0, GH)], idx_v.at[0], isem
                    ).wait()
                    gg = nrb // GH + 1

                    @pl.when(gg < NG)
                    def _():
                        pltpu.async_copy(
                            x_hbm.at[pl.ds(base + gg * GH, GH)],
                            idx_v.at[gg & 1],
                            isem,
                        )

            @pl.when(nh < 2 * BPW)
            def _():
                fire(nh, k)
        return carry

    lax.fori_loop(0, BPW // 2, body, 0)

    pltpu.sync_copy(h_v, out_hbm.at[pl.ds(base, BPW)])


def _fc_body(h_ref, w_ref, b_ref, o_ref):
    logits = lax.dot_general(
        h_ref[...], w_ref[...], (((1,), (1,)), ((), ())),
        preferred_element_type=jnp.float32,
    )
    o_ref[...] = jax.nn.sigmoid(logits + b_ref[...])


def _fc(h, W, b):
    return pl.pallas_call(
        _fc_body,
        out_shape=jax.ShapeDtypeStruct((B, NCLASS), jnp.float32),
    )(h, W, b.reshape(1, NCLASS))


def kernel(x, table, W, b):
    xi = x.astype(jnp.int32).reshape(B, CH, CHN)
    h = _pool(xi, table)
    return _fc(h, W, b)
